# Initial kernel scaffold; baseline (speedup 1.0000x reference)
import jax
import jax.numpy as jnp
from jax.experimental import pallas as pl


def kernel(z, edge_index, edge_type, weight):
    raise NotImplementedError("write your pallas kernel here")


# speedup vs baseline: 2.9979x; 2.9979x over previous
import functools

import jax
import jax.numpy as jnp
from jax import lax
from jax.experimental import pallas as pl
from jax.experimental.pallas import tpu as pltpu
from jax.experimental.pallas import tpu_sc as plsc

D = 128
L = 16
NC = 2
NS = 16
NW = NC * NS
CHUNK = 80


def _distmult(src_i, dst_i, rel_i, z, w):
    E = src_i.shape[0]
    epw = E // NW
    n_chunks = epw // CHUNK
    mesh = plsc.VectorSubcoreMesh(core_axis_name="c", subcore_axis_name="s")

    @functools.partial(
        pl.kernel,
        mesh=mesh,
        out_type=jax.ShapeDtypeStruct((E,), jnp.float32),
        compiler_params=pltpu.CompilerParams(needs_layout_passes=False),
        scratch_types=[
            pltpu.VMEM((CHUNK,), jnp.int32),
            pltpu.VMEM((CHUNK,), jnp.int32),
            pltpu.VMEM((CHUNK,), jnp.int32),
            pltpu.VMEM((CHUNK, D), jnp.float32),
            pltpu.VMEM((CHUNK, D), jnp.float32),
            pltpu.VMEM((CHUNK, D), jnp.float32),
            pltpu.VMEM((CHUNK,), jnp.float32),
            pltpu.SemaphoreType.DMA,
        ],
    )
    def k(src_hbm, dst_hbm, rel_hbm, z_hbm, w_hbm, out_hbm,
          si, di, ri, srows, drows, rrows, outv, sem):
        wid = lax.axis_index("s") * NC + lax.axis_index("c")
        base = wid * epw
        iota = lax.iota(jnp.int32, L)

        def chunk_body(c, carry):
            off = base + c * CHUNK
            pltpu.sync_copy(src_hbm.at[pl.ds(off, CHUNK)], si)
            pltpu.sync_copy(dst_hbm.at[pl.ds(off, CHUNK)], di)
            pltpu.sync_copy(rel_hbm.at[pl.ds(off, CHUNK)], ri)
            c1 = pltpu.async_copy(z_hbm.at[si], srows, sem)
            c2 = pltpu.async_copy(z_hbm.at[di], drows, sem)
            c3 = pltpu.async_copy(w_hbm.at[ri], rrows, sem)
            c1.wait()
            c2.wait()
            c3.wait()

            def group_body(g, carry2):
                tot = jnp.zeros((L,), jnp.float32)
                for e in range(L):
                    row = g * L + e
                    acc = jnp.zeros((L,), jnp.float32)
                    for kk in range(D // L):
                        s = srows[row, pl.ds(kk * L, L)]
                        t = drows[row, pl.ds(kk * L, L)]
                        r = rrows[row, pl.ds(kk * L, L)]
                        acc = acc + s * t * r
                    tot = jnp.where(iota == e, jnp.sum(acc), tot)
                outv[pl.ds(g * L, L)] = 1.0 / (1.0 + jnp.exp(-tot))
                return carry2

            lax.fori_loop(0, CHUNK // L, group_body, 0)
            pltpu.sync_copy(outv, out_hbm.at[pl.ds(off, CHUNK)])
            return carry

        lax.fori_loop(0, n_chunks, chunk_body, 0)

    return k(src_i, dst_i, rel_i, z, w)


def kernel(z, edge_index, edge_type, weight):
    ei = edge_index.astype(jnp.int32)
    et = edge_type.astype(jnp.int32)
    return _distmult(ei[0], ei[1], et, z, weight)


# speedup vs baseline: 8.3340x; 2.7800x over previous
import functools

import jax
import jax.numpy as jnp
from jax import lax
from jax.experimental import pallas as pl
from jax.experimental.pallas import tpu as pltpu
from jax.experimental.pallas import tpu_sc as plsc

D = 128
L = 16
NC = 2
NS = 16
NW = NC * NS
CHUNK = 80


def _distmult(src_i, dst_i, rel_i, z, w):
    E = src_i.shape[0]
    epw = E // NW
    n_chunks = epw // CHUNK
    n_pairs = n_chunks // 2
    mesh = plsc.VectorSubcoreMesh(core_axis_name="c", subcore_axis_name="s")

    @functools.partial(
        pl.kernel,
        mesh=mesh,
        out_type=jax.ShapeDtypeStruct((E,), jnp.float32),
        compiler_params=pltpu.CompilerParams(needs_layout_passes=False),
        scratch_types=[
            pltpu.VMEM((epw,), jnp.int32),
            pltpu.VMEM((epw,), jnp.int32),
            pltpu.VMEM((epw,), jnp.int32),
            pltpu.VMEM((CHUNK, D), jnp.float32),
            pltpu.VMEM((CHUNK, D), jnp.float32),
            pltpu.VMEM((CHUNK, D), jnp.float32),
            pltpu.VMEM((CHUNK, D), jnp.float32),
            pltpu.VMEM((CHUNK, D), jnp.float32),
            pltpu.VMEM((CHUNK, D), jnp.float32),
            pltpu.VMEM((L * L,), jnp.float32),
            pltpu.VMEM((epw,), jnp.float32),
            pltpu.SemaphoreType.DMA,
            pltpu.SemaphoreType.DMA,
        ],
    )
    def k(src_hbm, dst_hbm, rel_hbm, z_hbm, w_hbm, out_hbm,
          sidx, didx, ridx, s0, d0, r0, s1, d1, r1, tsc, outv,
          sem0, sem1):
        wid = lax.axis_index("s") * NC + lax.axis_index("c")
        base = wid * epw
        iota16 = lax.iota(jnp.int32, L) * L

        pltpu.sync_copy(src_hbm.at[pl.ds(base, epw)], sidx)
        pltpu.sync_copy(dst_hbm.at[pl.ds(base, epw)], didx)
        pltpu.sync_copy(rel_hbm.at[pl.ds(base, epw)], ridx)

        def issue(c, sb, db, rb, sem):
            off = c * CHUNK
            pltpu.async_copy(z_hbm.at[sidx.at[pl.ds(off, CHUNK)]], sb, sem)
            pltpu.async_copy(z_hbm.at[didx.at[pl.ds(off, CHUNK)]], db, sem)
            pltpu.async_copy(w_hbm.at[ridx.at[pl.ds(off, CHUNK)]], rb, sem)

        def drain(sb, db, rb, sem):
            dummy = sidx.at[pl.ds(0, CHUNK)]
            pltpu.make_async_copy(z_hbm.at[dummy], sb, sem).wait()
            pltpu.make_async_copy(z_hbm.at[dummy], db, sem).wait()
            pltpu.make_async_copy(z_hbm.at[dummy], rb, sem).wait()

        def compute(c, sb, db, rb):
            def group_body(g, carry):
                for e in range(L):
                    row = g * L + e
                    acc = jnp.zeros((L,), jnp.float32)
                    for kk in range(D // L):
                        s = sb[row, pl.ds(kk * L, L)]
                        t = db[row, pl.ds(kk * L, L)]
                        r = rb[row, pl.ds(kk * L, L)]
                        acc = acc + s * t * r
                    tsc[pl.ds(e * L, L)] = acc
                tot = jnp.zeros((L,), jnp.float32)
                for dd in range(L):
                    tot = tot + plsc.load_gather(tsc, [iota16 + dd])
                outv[pl.ds(c * CHUNK + g * L, L)] = 1.0 / (1.0 + jnp.exp(-tot))
                return carry

            lax.fori_loop(0, CHUNK // L, group_body, 0)

        issue(0, s0, d0, r0, sem0)

        def pair_body(p, carry):
            c0 = 2 * p
            issue(c0 + 1, s1, d1, r1, sem1)
            drain(s0, d0, r0, sem0)
            compute(c0, s0, d0, r0)
            issue(c0 + 2, s0, d0, r0, sem0)
            drain(s1, d1, r1, sem1)
            compute(c0 + 1, s1, d1, r1)
            return carry

        lax.fori_loop(0, n_pairs, pair_body, 0)
        drain(s0, d0, r0, sem0)
        compute(n_chunks - 1, s0, d0, r0)
        pltpu.sync_copy(outv, out_hbm.at[pl.ds(base, epw)])

    return k(src_i, dst_i, rel_i, z, w)


def kernel(z, edge_index, edge_type, weight):
    ei = edge_index.astype(jnp.int32)
    et = edge_type.astype(jnp.int32)
    return _distmult(ei[0], ei[1], et, z, weight)


# speedup vs baseline: 9.3908x; 1.1268x over previous
import functools

import jax
import jax.numpy as jnp
from jax import lax
from jax.experimental import pallas as pl
from jax.experimental.pallas import tpu as pltpu
from jax.experimental.pallas import tpu_sc as plsc

D = 128
L = 16
NC = 2
NS = 16
NW = NC * NS
CHUNK = 80


def _distmult(src_i, dst_i, rel_i, z, w):
    E = src_i.shape[0]
    epw = E // NW
    n_chunks = epw // CHUNK
    n_pairs = n_chunks // 2
    mesh = plsc.VectorSubcoreMesh(core_axis_name="c", subcore_axis_name="s")

    @functools.partial(
        pl.kernel,
        mesh=mesh,
        out_type=jax.ShapeDtypeStruct((E,), jnp.float32),
        compiler_params=pltpu.CompilerParams(
            needs_layout_passes=False, use_tc_tiling_on_sc=False),
        scratch_types=[
            pltpu.VMEM((epw,), jnp.int32),
            pltpu.VMEM((epw,), jnp.int32),
            pltpu.VMEM((epw,), jnp.int32),
            pltpu.VMEM((CHUNK, D // 2), jnp.int32),
            pltpu.VMEM((CHUNK, D // 2), jnp.int32),
            pltpu.VMEM((CHUNK, D // 2), jnp.int32),
            pltpu.VMEM((CHUNK, D // 2), jnp.int32),
            pltpu.VMEM((CHUNK, D // 2), jnp.int32),
            pltpu.VMEM((CHUNK, D // 2), jnp.int32),
            pltpu.VMEM((L * L,), jnp.float32),
            pltpu.VMEM((epw,), jnp.float32),
            pltpu.SemaphoreType.DMA,
            pltpu.SemaphoreType.DMA,
        ],
    )
    def k(src_hbm, dst_hbm, rel_hbm, z_hbm, w_hbm, out_hbm,
          sidx, didx, ridx, s0, d0, r0, s1, d1, r1, tsc, outv,
          sem0, sem1):
        wid = lax.axis_index("s") * NC + lax.axis_index("c")
        base = wid * epw
        iota16 = lax.iota(jnp.int32, L) * L

        pltpu.sync_copy(src_hbm.at[pl.ds(base, epw)], sidx)
        pltpu.sync_copy(dst_hbm.at[pl.ds(base, epw)], didx)
        pltpu.sync_copy(rel_hbm.at[pl.ds(base, epw)], ridx)

        def issue(c, sb, db, rb, sem):
            off = c * CHUNK
            pltpu.async_copy(z_hbm.at[sidx.at[pl.ds(off, CHUNK)]], sb, sem)
            pltpu.async_copy(z_hbm.at[didx.at[pl.ds(off, CHUNK)]], db, sem)
            pltpu.async_copy(w_hbm.at[ridx.at[pl.ds(off, CHUNK)]], rb, sem)

        def drain(sb, db, rb, sem):
            dummy = sidx.at[pl.ds(0, CHUNK)]
            pltpu.make_async_copy(z_hbm.at[dummy], sb, sem).wait()
            pltpu.make_async_copy(z_hbm.at[dummy], db, sem).wait()
            pltpu.make_async_copy(z_hbm.at[dummy], rb, sem).wait()

        def compute(c, sb, db, rb):
            def group_body(g, carry):
                for e in range(L):
                    row = g * L + e
                    acc0 = jnp.zeros((L,), jnp.float32)
                    acc1 = jnp.zeros((L,), jnp.float32)
                    for kk in range(D // (2 * L)):
                        s = plsc.bitcast(sb[row, pl.ds(kk * L, L)], jnp.bfloat16)
                        t = plsc.bitcast(db[row, pl.ds(kk * L, L)], jnp.bfloat16)
                        r = plsc.bitcast(rb[row, pl.ds(kk * L, L)], jnp.bfloat16)
                        pa, pb = plsc.unpack(
                            s * t * r, format=plsc.PackFormat.INTERLEAVED)
                        acc0 = acc0 + pa
                        acc1 = acc1 + pb
                    tsc[pl.ds(e * L, L)] = acc0 + acc1
                tot = jnp.zeros((L,), jnp.float32)
                for dd in range(L):
                    tot = tot + plsc.load_gather(tsc, [iota16 + dd])
                outv[pl.ds(c * CHUNK + g * L, L)] = 1.0 / (1.0 + jnp.exp(-tot))
                return carry

            lax.fori_loop(0, CHUNK // L, group_body, 0)

        issue(0, s0, d0, r0, sem0)

        def pair_body(p, carry):
            c0 = 2 * p
            issue(c0 + 1, s1, d1, r1, sem1)
            drain(s0, d0, r0, sem0)
            compute(c0, s0, d0, r0)
            issue(c0 + 2, s0, d0, r0, sem0)
            drain(s1, d1, r1, sem1)
            compute(c0 + 1, s1, d1, r1)
            return carry

        lax.fori_loop(0, n_pairs, pair_body, 0)
        drain(s0, d0, r0, sem0)
        compute(n_chunks - 1, s0, d0, r0)
        pltpu.sync_copy(outv, out_hbm.at[pl.ds(base, epw)])

    return k(src_i, dst_i, rel_i, z, w)


def kernel(z, edge_index, edge_type, weight):
    ei = edge_index.astype(jnp.int32)
    et = edge_type.astype(jnp.int32)
    zb = jax.lax.bitcast_convert_type(
        z.astype(jnp.bfloat16).reshape(z.shape[0], D // 2, 2), jnp.int32)
    wb = jax.lax.bitcast_convert_type(
        weight.astype(jnp.bfloat16).reshape(weight.shape[0], D // 2, 2),
        jnp.int32)
    return _distmult(ei[0], ei[1], et, zb, wb)
